# Initial kernel scaffold; baseline (speedup 1.0000x reference)
import jax
import jax.numpy as jnp
from jax.experimental import pallas as pl


def kernel(x, Wg, bg, W1, b1, W2, b2):
    raise NotImplementedError("write your pallas kernel here")


# speedup vs baseline: 2.3243x; 2.3243x over previous
import functools

import jax
import jax.numpy as jnp
from jax.experimental import pallas as pl
import jax.experimental.pallas.tpu as pltpu

E = 8
D = 768
H = 768
N = 2048
B = 256
NBLK = N // B + E


def _gate_kernel(x_ref, wg_ref, bg_ref,
                 perm_ref, counts_ref, avgp_ref, aux_ref, var_ref):
    x = x_ref[:]
    wg = wg_ref[:]
    logits = jax.lax.dot_general(
        x.astype(jnp.bfloat16), wg.astype(jnp.bfloat16),
        (((1,), (1,)), ((), ())),
        preferred_element_type=jnp.float32) + bg_ref[:]
    probs = jax.nn.softmax(logits, axis=-1)
    p_top = jnp.max(probs, axis=-1, keepdims=True)
    e_iota = jax.lax.broadcasted_iota(jnp.int32, (N, E), 1)
    cand = jnp.where(probs == p_top, e_iota, E)
    top1 = jnp.min(cand, axis=-1, keepdims=True)
    onehot = (e_iota == top1).astype(jnp.float32)

    counts = jnp.sum(onehot, axis=0, keepdims=True)
    counts_ref[:] = counts
    sum_p = jnp.sum(p_top * onehot, axis=0, keepdims=True)
    avgp_ref[:] = jnp.where(counts > 0, sum_p / jnp.maximum(counts, 1.0), 0.0)

    csum = onehot
    shift = 1
    while shift < N:
        csum = csum + jnp.concatenate(
            [jnp.zeros((shift, E), jnp.float32), csum[:-shift, :]], axis=0)
        shift *= 2
    ccum = counts
    shift = 1
    while shift < E:
        ccum = ccum + jnp.concatenate(
            [jnp.zeros((1, shift), jnp.float32), ccum[:, :-shift]], axis=1)
        shift *= 2
    offsets = ccum - counts
    rank = jnp.sum(csum * onehot, axis=-1, keepdims=True) - 1.0
    pos = jnp.sum(offsets * onehot, axis=-1, keepdims=True) + rank

    tok_ids = jax.lax.broadcasted_iota(jnp.int32, (N, 1), 0).astype(jnp.float32)
    for j in range(N // B):
        slots = jax.lax.broadcasted_iota(
            jnp.int32, (1, B), 1).astype(jnp.float32) + (j * B)
        P = (pos == slots).astype(jnp.float32)
        perm_ref[j:j + 1, :] = jnp.sum(
            P * tok_ids, axis=0, keepdims=True).astype(jnp.int32)

    avg_usage = counts / N
    aux_ref[:] = jnp.sum(avg_usage * avg_usage, keepdims=True) * E
    mean_scores = jnp.mean(probs, axis=0, keepdims=True)
    diff = probs - mean_scores
    var_ref[:] = -jnp.mean(diff * diff, keepdims=True)


def _ffn_kernel(s_e, s_base, s_valid, s_first, s_last, s_perm, s_avgp, s_deninv,
                x_ref, w1_ref, b1_ref, w2_ref, b2_ref,
                out_ref, orth_ref,
                xg_ref, so_ref, g_ref, dsq_ref):
    g = pl.program_id(0)

    @pl.when(g == 0)
    def _():
        orth_ref[:] = jnp.zeros((1, 1), jnp.float32)

    valid = s_valid[g]

    @pl.when(valid > 0)
    def _():
        base = s_base[g]
        e = s_e[g]

        def gather_body(r, c):
            tok = s_perm[jnp.minimum(base + r, N - 1)]
            xg_ref[pl.ds(r, 1), :] = x_ref[pl.ds(tok, 1), :]
            return c
        jax.lax.fori_loop(0, B, gather_body, 0, unroll=8)

        rows = jax.lax.broadcasted_iota(jnp.int32, (B, 1), 0)
        xv = xg_ref[:]
        h = jnp.maximum(
            jax.lax.dot_general(xv, w1_ref[0], (((1,), (1,)), ((), ())),
                                preferred_element_type=jnp.float32)
            + b1_ref[0], 0.0)
        outv = jax.lax.dot_general(h, w2_ref[0], (((1,), (1,)), ((), ())),
                                   preferred_element_type=jnp.float32) \
            + b2_ref[0]
        m = jnp.where(rows < valid, outv, 0.0)

        gblk = jax.lax.dot_general(m, m, (((0,), (0,)), ((), ())),
                                   preferred_element_type=jnp.float32)
        dvals = jnp.sum(m * m, axis=-1, keepdims=True)
        dq = jnp.sum(dvals * dvals, keepdims=True)

        @pl.when(s_first[g] == 1)
        def _():
            g_ref[:] = gblk
            dsq_ref[:] = dq

        @pl.when(s_first[g] == 0)
        def _():
            g_ref[:] = g_ref[:] + gblk
            dsq_ref[:] = dsq_ref[:] + dq

        @pl.when(s_last[g] == 1)
        def _():
            gv = g_ref[:]
            ssq = jnp.sum(gv * gv, keepdims=True)
            orth_ref[:] = orth_ref[:] + (ssq - dsq_ref[:]) * s_deninv[e]

        so_ref[:] = s_avgp[e] * m

        def scatter_body(r, c):
            @pl.when(r < valid)
            def _():
                tok = s_perm[jnp.minimum(base + r, N - 1)]
                out_ref[pl.ds(tok, 1), :] = so_ref[pl.ds(r, 1), :]
            return c
        jax.lax.fori_loop(0, B, scatter_body, 0, unroll=8)


@jax.jit
def kernel(x, Wg, bg, W1, b1, W2, b2):
    perm2d, counts, avgp, aux, var = pl.pallas_call(
        _gate_kernel,
        out_shape=[
            jax.ShapeDtypeStruct((N // B, B), jnp.int32),
            jax.ShapeDtypeStruct((1, E), jnp.float32),
            jax.ShapeDtypeStruct((1, E), jnp.float32),
            jax.ShapeDtypeStruct((1, 1), jnp.float32),
            jax.ShapeDtypeStruct((1, 1), jnp.float32),
        ],
    )(x, Wg, bg.reshape(1, E))

    perm = perm2d.reshape(N)
    counts1 = counts.reshape(E)
    counts_i = counts1.astype(jnp.int32)
    nblk = (counts_i + B - 1) // B
    cum = jnp.cumsum(nblk)
    gids = jnp.arange(NBLK, dtype=jnp.int32)
    e_of = jnp.sum((gids[:, None] >= cum[None, :]).astype(jnp.int32), axis=-1)
    last_active = jnp.max(jnp.where(nblk > 0, jnp.arange(E, dtype=jnp.int32), 0))
    e_of = jnp.minimum(e_of, last_active)
    start_of = cum[e_of] - nblk[e_of]
    j = gids - start_of
    offsets = jnp.cumsum(counts_i) - counts_i
    base = jnp.clip(offsets[e_of] + j * B, 0, N - 1)
    validv = jnp.clip(counts_i[e_of] - j * B, 0, B)
    first = ((j == 0) & (validv > 0)).astype(jnp.int32)
    last = ((j == nblk[e_of] - 1) & (validv > 0)).astype(jnp.int32)
    deninv = 1.0 / jnp.maximum(counts1 * counts1, 1.0)

    grid_spec = pltpu.PrefetchScalarGridSpec(
        num_scalar_prefetch=8,
        grid=(NBLK,),
        in_specs=[
            pl.BlockSpec((N, D), lambda g, *s: (0, 0)),
            pl.BlockSpec((1, H, D), lambda g, *s: (s[0][g], 0, 0)),
            pl.BlockSpec((1, 1, H), lambda g, *s: (s[0][g], 0, 0)),
            pl.BlockSpec((1, D, H), lambda g, *s: (s[0][g], 0, 0)),
            pl.BlockSpec((1, 1, D), lambda g, *s: (s[0][g], 0, 0)),
        ],
        out_specs=[
            pl.BlockSpec((N, D), lambda g, *s: (0, 0)),
            pl.BlockSpec((1, 1), lambda g, *s: (0, 0)),
        ],
        scratch_shapes=[
            pltpu.VMEM((B, D), jnp.float32),
            pltpu.VMEM((B, D), jnp.float32),
            pltpu.VMEM((D, D), jnp.float32),
            pltpu.VMEM((1, 1), jnp.float32),
        ],
    )
    out, orth = pl.pallas_call(
        _ffn_kernel,
        grid_spec=grid_spec,
        out_shape=[
            jax.ShapeDtypeStruct((N, D), jnp.float32),
            jax.ShapeDtypeStruct((1, 1), jnp.float32),
        ],
        compiler_params=pltpu.CompilerParams(
            dimension_semantics=("arbitrary",),
        ),
    )(e_of, base, validv, first, last, perm, avgp.reshape(E), deninv,
      x, W1, b1.reshape(E, 1, H), W2, b2.reshape(E, 1, D))

    total = (0.001 * aux.reshape(()) + 0.001 * orth.reshape(())
             + 0.001 * var.reshape(()))
    return out, total
